# baseline (speedup 1.0000x reference)
import functools

import jax
import jax.numpy as jnp
from jax import lax
from jax.experimental import pallas as pl
from jax.experimental.pallas import tpu as pltpu
from jax.experimental.pallas import tpu_sc as plsc

_NUM_CORES = 2
_NUM_SUBCORES = 16
_NW = _NUM_CORES * _NUM_SUBCORES
_LANES = 16


def _sc_fused(x, table_flat, idx2, nrd):
    B, H, S1, S2 = x.shape
    spad, W = idx2.shape
    ngrp = spad // 8
    nvec = W // _LANES
    nfull = S2 // _LANES
    tail = S2 - nfull * _LANES
    nrounds = -(-ngrp // _NW)
    mesh = plsc.VectorSubcoreMesh(core_axis_name="c", subcore_axis_name="s")

    @functools.partial(
        pl.kernel,
        out_type=jax.ShapeDtypeStruct((B, H, S1, S2), jnp.float32),
        mesh=mesh,
        compiler_params=pltpu.CompilerParams(needs_layout_passes=False),
        scratch_types=[
            pltpu.VMEM((8, W), jnp.int32),
            pltpu.VMEM((H * nrd,), jnp.float32),
            pltpu.VMEM((8, W), jnp.float32),
            [pltpu.VMEM((8, S2), jnp.float32) for _ in range(B)],
            [pltpu.SemaphoreType.DMA for _ in range(B)],
            [pltpu.SemaphoreType.DMA for _ in range(B)],
        ],
    )
    def k(x_hbm, table_hbm, idx_hbm, out_hbm, idx_v, tab_v, pbuf, xbufs,
          sems_in, sems_out):
        wid = lax.axis_index("s") * _NUM_CORES + lax.axis_index("c")
        pltpu.sync_copy(table_hbm, tab_v)

        def do_group(rg):
            r0 = rg * 8
            pltpu.sync_copy(idx_hbm.at[pl.ds(r0, 8), :], idx_v)
            pending_out = [None] * B
            for h in range(H):
                ins = []
                for b in range(B):
                    if pending_out[b] is not None:
                        pending_out[b].wait()
                        pending_out[b] = None
                    ins.append(
                        pltpu.async_copy(
                            x_hbm.at[b, h, pl.ds(r0, 8), :],
                            xbufs[b],
                            sems_in[b],
                        )
                    )
                hoff = jnp.full((_LANES,), h * nrd, jnp.int32)

                def g_row(r, c):
                    def g_vec(j, c2):
                        sl = pl.ds(j * _LANES, _LANES)
                        pbuf[r, sl] = plsc.load_gather(
                            tab_v, [idx_v[r, sl] + hoff]
                        )
                        return c2

                    return lax.fori_loop(0, nvec, g_vec, c)

                lax.fori_loop(0, 8, g_row, 0)

                for b in range(B):
                    ins[b].wait()
                    xbuf = xbufs[b]

                    def a_row(r, c):
                        def a_vec(j, c2):
                            sl = pl.ds(j * _LANES, _LANES)
                            xbuf[r, sl] = xbuf[r, sl] + pbuf[r, sl]
                            return c2

                        lax.fori_loop(0, nfull, a_vec, c)
                        if tail:
                            rv = jnp.full((_LANES,), r, jnp.int32)
                            cv = nfull * _LANES + lax.iota(jnp.int32, _LANES)
                            m = cv < S2
                            xv = plsc.load_gather(xbuf, [rv, cv], mask=m)
                            pv = pbuf[r, pl.ds(nfull * _LANES, _LANES)]
                            plsc.store_scatter(xbuf, [rv, cv], xv + pv, mask=m)
                        return c

                    lax.fori_loop(0, 8, a_row, 0)
                    pending_out[b] = pltpu.async_copy(
                        xbuf,
                        out_hbm.at[b, h, pl.ds(r0, 8), :],
                        sems_out[b],
                    )
            for b in range(B):
                pending_out[b].wait()

        def round_body(rnd, c):
            rg = wid + rnd * _NW

            @pl.when(rg < ngrp)
            def _():
                do_group(rg)

            return c

        lax.fori_loop(0, nrounds, round_body, 0)

    return k(x, table_flat, idx2)


def kernel(inputs, positional_embedding, relative_position_index):
    B, H, S1, S2 = inputs.shape
    idx = relative_position_index[:S1, :S2]
    nrd = positional_embedding.shape[1]
    spad = -(-S1 // 8) * 8
    W = -(-S2 // _LANES) * _LANES
    idx2 = jnp.pad(idx.astype(jnp.int32), ((0, spad - S1), (0, W - S2)))
    table_flat = jnp.reshape(positional_embedding, (H * nrd,))
    return _sc_fused(inputs, table_flat, idx2, nrd)


# speedup vs baseline: 2.0020x; 2.0020x over previous
import functools

import jax
import jax.numpy as jnp
from jax import lax
from jax.experimental import pallas as pl
from jax.experimental.pallas import tpu as pltpu
from jax.experimental.pallas import tpu_sc as plsc

_NUM_CORES = 2
_NUM_SUBCORES = 16
_NW = _NUM_CORES * _NUM_SUBCORES
_LANES = 16


def _sc_gather(table_flat, idx2, H, nrd):
    spad, W = idx2.shape
    nvec = W // _LANES
    main_rows = 16 * _NW
    rest = spad - min(spad, main_rows)
    assert rest % 8 == 0 and rest // 8 <= _NW
    mesh = plsc.VectorSubcoreMesh(core_axis_name="c", subcore_axis_name="s")

    @functools.partial(
        pl.kernel,
        out_type=jax.ShapeDtypeStruct((H, spad, W), jnp.float32),
        mesh=mesh,
        compiler_params=pltpu.CompilerParams(needs_layout_passes=False),
        scratch_types=[
            pltpu.VMEM((16, W), jnp.int32),
            pltpu.VMEM((H * nrd,), jnp.float32),
            pltpu.VMEM((16, W), jnp.float32),
            pltpu.VMEM((16, W), jnp.float32),
            pltpu.SemaphoreType.DMA,
            pltpu.SemaphoreType.DMA,
        ],
    )
    def k(table_hbm, idx_hbm, out_hbm, idx_v, tab_v, buf0, buf1, sem0, sem1):
        wid = lax.axis_index("s") * _NUM_CORES + lax.axis_index("c")
        pltpu.sync_copy(table_hbm, tab_v)
        bufs = (buf0, buf1)
        sems = (sem0, sem1)

        def do_span(r0, R):
            pltpu.sync_copy(idx_hbm.at[pl.ds(r0, R), :], idx_v.at[pl.ds(0, R), :])
            pending = [None, None]
            for h in range(H):
                buf, sem = bufs[h % 2], sems[h % 2]
                if pending[h % 2] is not None:
                    pending[h % 2].wait()
                hoff = jnp.full((_LANES,), h * nrd, jnp.int32)

                def g_row(r, c):
                    def g_vec(j, c2):
                        sl = pl.ds(j * _LANES, _LANES)
                        buf[r, sl] = plsc.load_gather(
                            tab_v, [idx_v[r, sl] + hoff]
                        )
                        return c2

                    return lax.fori_loop(0, nvec, g_vec, c)

                lax.fori_loop(0, R, g_row, 0)
                pending[h % 2] = pltpu.async_copy(
                    buf.at[pl.ds(0, R), :],
                    out_hbm.at[h, pl.ds(r0, R), :],
                    sem,
                )
            for p in pending:
                p.wait()

        if main_rows <= spad:
            do_span(wid * 16, 16)
        else:
            @pl.when(wid * 16 < spad)
            def _():
                do_span(wid * 16, 16)
        if rest:

            @pl.when(wid < rest // 8)
            def _():
                do_span(main_rows + wid * 8, 8)

    return k(table_flat, idx2)


def _tc_add(inputs, pos3):
    B, H, S1, S2 = inputs.shape
    _, spad, W = pos3.shape

    def body(x_ref, p_ref, o_ref):
        o_ref[:, 0] = x_ref[:, 0] + p_ref[:1, :S1, :S2]

    return pl.pallas_call(
        body,
        grid=(H,),
        in_specs=[
            pl.BlockSpec((B, 1, S1, S2), lambda h: (0, h, 0, 0)),
            pl.BlockSpec((1, spad, W), lambda h: (h, 0, 0)),
        ],
        out_specs=pl.BlockSpec((B, 1, S1, S2), lambda h: (0, h, 0, 0)),
        out_shape=jax.ShapeDtypeStruct((B, H, S1, S2), jnp.float32),
    )(inputs, pos3)


def kernel(inputs, positional_embedding, relative_position_index):
    B, H, S1, S2 = inputs.shape
    idx = relative_position_index[:S1, :S2]
    nrd = positional_embedding.shape[1]
    spad = -(-S1 // 8) * 8
    W = -(-S2 // _LANES) * _LANES
    idx2 = jnp.pad(idx.astype(jnp.int32), ((0, spad - S1), (0, W - S2)))
    table_flat = jnp.reshape(positional_embedding, (H * nrd,))
    pos3 = _sc_gather(table_flat, idx2, H, nrd)
    return _tc_add(inputs, pos3)
